# Initial kernel scaffold; baseline (speedup 1.0000x reference)
import jax
import jax.numpy as jnp
from jax.experimental import pallas as pl


def kernel(x, Wg, bg, W1, b1, W2, b2):
    raise NotImplementedError("write your pallas kernel here")


# speedup vs baseline: 1.1885x; 1.1885x over previous
import functools

import jax
import jax.numpy as jnp
from jax import lax
from jax.experimental import pallas as pl
from jax.experimental.pallas import tpu as pltpu

_N = 2048
_D = 1024
_DFF = 2048
_G0, _G1 = 2, 4
_E = _G0 * _G1
_TM = 256
_NT = _N // _TM
_NEG = -1e30


def _gate_body(x_ref, wg_ref, bg_ref, w_ref):
    scores = jnp.dot(x_ref[...], wg_ref[...], preferred_element_type=jnp.float32)
    scores = scores + bg_ref[...]
    s0 = scores[:, :_G0]
    s1 = scores[:, _G0:_G0 + _G1]
    logits = jnp.concatenate([s0[:, i:i + 1] + s1 for i in range(_G0)], axis=1)
    ei = lax.broadcasted_iota(jnp.int32, (_TM, _E), 1)
    m1 = jnp.max(logits, axis=1, keepdims=True)
    a1 = jnp.min(jnp.where(logits == m1, ei, _E), axis=1, keepdims=True)
    mask1 = ei == a1
    l2 = jnp.where(mask1, _NEG, logits)
    m2 = jnp.max(l2, axis=1, keepdims=True)
    a2 = jnp.min(jnp.where(l2 == m2, ei, _E), axis=1, keepdims=True)
    mask2 = ei == a2
    z = jnp.exp(m2 - m1)
    p1 = 1.0 / (1.0 + z)
    p2 = z / (1.0 + z)
    w_ref[...] = jnp.where(mask1, p1, 0.0) + jnp.where(mask2, p2, 0.0)


def _moe_body(xb_ref, w1_ref, b1_ref, w2_ref, b2_ref, wden_ref, out_ref):
    e = pl.program_id(0)
    t = pl.program_id(1)
    h = jnp.dot(xb_ref[...], w1_ref[0], preferred_element_type=jnp.float32)
    h = jnp.maximum(h + b1_ref[0], 0.0)
    y = jnp.dot(h.astype(jnp.bfloat16), w2_ref[0],
                preferred_element_type=jnp.float32)
    y = y + b2_ref[0]
    ei = lax.broadcasted_iota(jnp.int32, (_TM, _E), 1)
    wcol = jnp.sum(jnp.where(ei == e, wden_ref[...], 0.0), axis=1,
                   keepdims=True)
    contrib = wcol * y
    rows = pl.ds(t * _TM, _TM)

    @pl.when(e == 0)
    def _init():
        out_ref[rows, :] = contrib

    @pl.when(e > 0)
    def _acc():
        out_ref[rows, :] = out_ref[rows, :] + contrib


def kernel(x, Wg, bg, W1, b1, W2, b2):
    gs = _G0 + _G1
    wden = pl.pallas_call(
        _gate_body,
        grid=(_NT,),
        in_specs=[
            pl.BlockSpec((_TM, _D), lambda t: (t, 0)),
            pl.BlockSpec((_D, gs), lambda t: (0, 0)),
            pl.BlockSpec((1, gs), lambda t: (0, 0)),
        ],
        out_specs=pl.BlockSpec((_TM, _E), lambda t: (t, 0)),
        out_shape=jax.ShapeDtypeStruct((_N, _E), jnp.float32),
    )(x, Wg, bg.reshape(1, gs))

    xb = x.astype(jnp.bfloat16)
    w1b = W1.astype(jnp.bfloat16)
    w2b = W2.astype(jnp.bfloat16)
    out = pl.pallas_call(
        _moe_body,
        grid=(_E, _NT),
        in_specs=[
            pl.BlockSpec((_TM, _D), lambda e, t: (t, 0)),
            pl.BlockSpec((1, _D, _DFF), lambda e, t: (e, 0, 0)),
            pl.BlockSpec((1, 1, _DFF), lambda e, t: (e, 0, 0)),
            pl.BlockSpec((1, _DFF, _D), lambda e, t: (e, 0, 0)),
            pl.BlockSpec((1, 1, _D), lambda e, t: (e, 0, 0)),
            pl.BlockSpec((_TM, _E), lambda e, t: (t, 0)),
        ],
        out_specs=pl.BlockSpec((_N, _D), lambda e, t: (0, 0)),
        out_shape=jax.ShapeDtypeStruct((_N, _D), jnp.float32),
        compiler_params=pltpu.CompilerParams(
            dimension_semantics=("arbitrary", "arbitrary"),
        ),
    )(xb, w1b, b1.reshape(_E, 1, _DFF), w2b, b2.reshape(_E, 1, _D), wden)
    return out


# speedup vs baseline: 1.4742x; 1.2404x over previous
import functools

import jax
import jax.numpy as jnp
from jax import lax
from jax.experimental import pallas as pl
from jax.experimental.pallas import tpu as pltpu
from jax.experimental.pallas import tpu_sc as plsc

_N = 2048
_D = 1024
_DFF = 2048
_G0, _G1 = 2, 4
_E = _G0 * _G1
_K = 2
_A = _N * _K
_TM = 256
_NT_X = _N // _TM
_NT_A = _A // _TM
_STEPS = _NT_A + _E - 1
_NEG = -1e30

_NC, _NS, _L = 2, 16, 16
_NW = _NC * _NS
_TPW = _N // _NW
_APW = _TPW * _K
_CPW = _APW // _L


def _gate_body(x_ref, wg_ref, bg_ref, eidx_ref, probs_ref):
    scores = jnp.dot(x_ref[...], wg_ref[...], preferred_element_type=jnp.float32)
    scores = scores + bg_ref[...]
    s0 = scores[:, :_G0]
    s1 = scores[:, _G0:_G0 + _G1]
    logits = jnp.concatenate([s0[:, i:i + 1] + s1 for i in range(_G0)], axis=1)
    ei = lax.broadcasted_iota(jnp.int32, (_TM, _E), 1)
    m1 = jnp.max(logits, axis=1, keepdims=True)
    a1 = jnp.min(jnp.where(logits == m1, ei, _E), axis=1, keepdims=True)
    l2 = jnp.where(ei == a1, _NEG, logits)
    m2 = jnp.max(l2, axis=1, keepdims=True)
    a2 = jnp.min(jnp.where(l2 == m2, ei, _E), axis=1, keepdims=True)
    z = jnp.exp(m2 - m1)
    p1 = 1.0 / (1.0 + z)
    p2 = z / (1.0 + z)
    eidx_ref[...] = jnp.concatenate([a1, a2], axis=1)
    probs_ref[...] = jnp.concatenate([p1, p2], axis=1)


def _gate(x, Wg, bg):
    gs = _G0 + _G1
    return pl.pallas_call(
        _gate_body,
        grid=(_NT_X,),
        in_specs=[
            pl.BlockSpec((_TM, _D), lambda t: (t, 0)),
            pl.BlockSpec((_D, gs), lambda t: (0, 0)),
            pl.BlockSpec((1, gs), lambda t: (0, 0)),
        ],
        out_specs=[
            pl.BlockSpec((_TM, _K), lambda t: (t, 0)),
            pl.BlockSpec((_TM, _K), lambda t: (t, 0)),
        ],
        out_shape=[
            jax.ShapeDtypeStruct((_N, _K), jnp.int32),
            jax.ShapeDtypeStruct((_N, _K), jnp.float32),
        ],
    )(x, Wg, bg.reshape(1, gs))


def _route_body(x_hbm, ef_hbm, a_hbm, pos_hbm, gs_hbm,
                ef_v, xrows_v, pe_v, po_v, gs_v, sem):
    wid = lax.axis_index("s") * _NC + lax.axis_index("c")
    iot = lax.iota(jnp.int32, _L)
    onehot = [(iot == e).astype(jnp.int32) for e in range(_E)]
    cpw_k = _TPW // _L

    pltpu.sync_copy(ef_hbm, ef_v)

    nchunks = _A // _L
    my0 = wid * cpw_k
    my1 = _N // _L + wid * cpw_k

    def hist_step(c, carry):
        tot, pre0, pre1 = carry
        chunk = ef_v[pl.ds(c * _L, _L)]
        b0 = (c < my0).astype(jnp.int32)
        b1 = (c < my1).astype(jnp.int32)
        for e in range(_E):
            cnt = jnp.sum((chunk == e).astype(jnp.int32))
            tot = tot + onehot[e] * cnt
            pre0 = pre0 + onehot[e] * (cnt * b0)
            pre1 = pre1 + onehot[e] * (cnt * b1)
        return tot, pre0, pre1

    zeros = jnp.zeros((_L,), jnp.int32)
    tot, pre0, pre1 = lax.fori_loop(0, nchunks, hist_step,
                                    (zeros, zeros, zeros))

    excl = jnp.cumsum(tot) - tot
    base0 = excl + pre0
    base1 = excl + pre1

    ch0 = [ef_v[pl.ds((my0 + c) * _L, _L)] for c in range(cpw_k)]
    ch1 = [ef_v[pl.ds((my1 + c) * _L, _L)] for c in range(cpw_k)]
    pc0 = [jnp.zeros((_L,), jnp.int32) for _ in range(cpw_k)]
    pc1 = [jnp.zeros((_L,), jnp.int32) for _ in range(cpw_k)]
    for e in range(_E):
        off0 = jnp.sum(jnp.where(onehot[e] == 1, base0, 0))
        off1 = jnp.sum(jnp.where(onehot[e] == 1, base1, 0))
        c0 = jnp.zeros((), jnp.int32)
        c1 = jnp.zeros((), jnp.int32)
        for c in range(cpw_k):
            m = ch0[c] == e
            rank = jnp.cumsum(m.astype(jnp.int32)) - 1
            pc0[c] = jnp.where(m, off0 + c0 + rank, pc0[c])
            c0 = c0 + jnp.sum(m.astype(jnp.int32))
            m = ch1[c] == e
            rank = jnp.cumsum(m.astype(jnp.int32)) - 1
            pc1[c] = jnp.where(m, off1 + c1 + rank, pc1[c])
            c1 = c1 + jnp.sum(m.astype(jnp.int32))
    for c in range(cpw_k):
        pe_v[pl.ds(c * _L, _L)] = pc0[c]
        po_v[pl.ds(c * _L, _L)] = pc1[c]

    pltpu.sync_copy(x_hbm.at[pl.ds(wid * _TPW, _TPW)], xrows_v)
    pltpu.async_copy(xrows_v, a_hbm.at[pe_v], sem).wait()
    pltpu.async_copy(xrows_v, a_hbm.at[po_v], sem).wait()

    pltpu.sync_copy(pe_v, pos_hbm.at[pl.ds(wid * _TPW, _TPW)])
    pltpu.sync_copy(po_v, pos_hbm.at[pl.ds(_N + wid * _TPW, _TPW)])

    @pl.when(wid == 0)
    def _():
        gs_v[...] = tot
        pltpu.sync_copy(gs_v, gs_hbm)


def _route(x, ef):
    mesh = plsc.VectorSubcoreMesh(core_axis_name="c", subcore_axis_name="s",
                                  num_cores=_NC, num_subcores=_NS)
    f = pl.kernel(
        _route_body,
        out_type=[
            jax.ShapeDtypeStruct((_A, _D), jnp.float32),
            jax.ShapeDtypeStruct((_A,), jnp.int32),
            jax.ShapeDtypeStruct((_L,), jnp.int32),
        ],
        mesh=mesh,
        compiler_params=pltpu.CompilerParams(needs_layout_passes=False),
        scratch_types=[
            pltpu.VMEM((_A,), jnp.int32),
            pltpu.VMEM((_TPW, _D), jnp.float32),
            pltpu.VMEM((_TPW,), jnp.int32),
            pltpu.VMEM((_TPW,), jnp.int32),
            pltpu.VMEM((_L,), jnp.int32),
            pltpu.SemaphoreType.DMA,
        ],
    )
    return f(x, ef)


def _ffn_body(p_ref, a_ref, w1_ref, b1_ref, w2_ref, b2_ref, out_ref):
    s = pl.program_id(0)
    lo = p_ref[2, s]
    hi = p_ref[3, s]
    first = p_ref[4, s]
    a = a_ref[...].astype(jnp.bfloat16)
    h = jnp.dot(a, w1_ref[0], preferred_element_type=jnp.float32)
    h = jnp.maximum(h + b1_ref[0], 0.0)
    y = jnp.dot(h.astype(jnp.bfloat16), w2_ref[0],
                preferred_element_type=jnp.float32)
    y = y + b2_ref[0]
    ri = lax.broadcasted_iota(jnp.int32, (_TM, _D), 0)
    contrib = jnp.where((ri >= lo) & (ri < hi), y, 0.0)

    @pl.when(first == 1)
    def _init():
        out_ref[...] = contrib

    @pl.when(first == 0)
    def _acc():
        out_ref[...] = out_ref[...] + contrib


def _tile_maps(gsz):
    offs = jnp.concatenate([jnp.zeros((1,), jnp.int32), jnp.cumsum(gsz)])
    t = jnp.arange(_NT_A, dtype=jnp.int32)[:, None]
    e = jnp.arange(_E, dtype=jnp.int32)[None, :]
    seg_lo = jnp.maximum(offs[:-1][None, :], t * _TM)
    seg_hi = jnp.minimum(offs[1:][None, :], (t + 1) * _TM)
    active = seg_hi > seg_lo
    key = jnp.where(active, t * _E + e, 32767).reshape(-1)
    order = jnp.argsort(key)[:_STEPS]
    act = key[order] < 32767
    t23 = jnp.where(act, order // _E, _NT_A - 1).astype(jnp.int32)
    e23 = jnp.where(act, order % _E, _E - 1).astype(jnp.int32)
    lo23 = jnp.where(act, seg_lo.reshape(-1)[order] - t23 * _TM, 0)
    hi23 = jnp.where(act, seg_hi.reshape(-1)[order] - t23 * _TM, 0)
    prev_t = jnp.concatenate([jnp.full((1,), -1, jnp.int32), t23[:-1]])
    first = (act & (t23 != prev_t)).astype(jnp.int32)
    return jnp.stack([t23, e23, lo23.astype(jnp.int32),
                      hi23.astype(jnp.int32), first])


def _ffn(a_sorted, pmap, w1b, b1, w2b, b2):
    grid_spec = pltpu.PrefetchScalarGridSpec(
        num_scalar_prefetch=1,
        grid=(_STEPS,),
        in_specs=[
            pl.BlockSpec((_TM, _D), lambda s, p: (p[0, s], 0)),
            pl.BlockSpec((1, _D, _DFF), lambda s, p: (p[1, s], 0, 0)),
            pl.BlockSpec((1, 1, _DFF), lambda s, p: (p[1, s], 0, 0)),
            pl.BlockSpec((1, _DFF, _D), lambda s, p: (p[1, s], 0, 0)),
            pl.BlockSpec((1, 1, _D), lambda s, p: (p[1, s], 0, 0)),
        ],
        out_specs=pl.BlockSpec((_TM, _D), lambda s, p: (p[0, s], 0)),
    )
    return pl.pallas_call(
        _ffn_body,
        grid_spec=grid_spec,
        out_shape=jax.ShapeDtypeStruct((_A, _D), jnp.float32),
        compiler_params=pltpu.CompilerParams(
            dimension_semantics=("arbitrary",),
        ),
    )(pmap, a_sorted, w1b, b1.reshape(_E, 1, _DFF), w2b,
      b2.reshape(_E, 1, _D))


def _combine_body(y_hbm, pos_hbm, probs_hbm, out_hbm,
                  peb_v, pob_v, pr0_v, pr1_v, rows_v, outs_v, sem):
    wid = lax.axis_index("s") * _NC + lax.axis_index("c")
    pltpu.sync_copy(pos_hbm.at[pl.ds(wid * _TPW, _TPW)], peb_v)
    pltpu.sync_copy(pos_hbm.at[pl.ds(_N + wid * _TPW, _TPW)], pob_v)
    pltpu.sync_copy(probs_hbm.at[pl.ds(wid * _TPW, _TPW)], pr0_v)
    pltpu.sync_copy(probs_hbm.at[pl.ds(_N + wid * _TPW, _TPW)], pr1_v)
    half = _TPW // 2
    for hh in range(2):
        pltpu.async_copy(y_hbm.at[peb_v.at[pl.ds(hh * half, half)]],
                         rows_v.at[pl.ds(0, half)], sem).wait()
        pltpu.async_copy(y_hbm.at[pob_v.at[pl.ds(hh * half, half)]],
                         rows_v.at[pl.ds(half, half)], sem).wait()
        p0s, p1s = [], []
        for g in range(half // _L):
            v0 = pr0_v[pl.ds(hh * half + g * _L, _L)]
            v1 = pr1_v[pl.ds(hh * half + g * _L, _L)]
            for j in range(_L):
                p0s.append(v0[j])
                p1s.append(v1[j])

        def chunk_step(cc, _):
            sl = pl.ds(cc * _L, _L)
            for i in range(half):
                outs_v[i, sl] = (p0s[i] * rows_v[i, sl]
                                 + p1s[i] * rows_v[half + i, sl])
            return 0

        lax.fori_loop(0, _D // _L, chunk_step, 0)
        pltpu.sync_copy(outs_v,
                        out_hbm.at[pl.ds(wid * _TPW + hh * half, half)])


def _combine(y, pos, probs_t):
    mesh = plsc.VectorSubcoreMesh(core_axis_name="c", subcore_axis_name="s",
                                  num_cores=_NC, num_subcores=_NS)
    f = pl.kernel(
        _combine_body,
        out_type=jax.ShapeDtypeStruct((_N, _D), jnp.float32),
        mesh=mesh,
        compiler_params=pltpu.CompilerParams(needs_layout_passes=False),
        scratch_types=[
            pltpu.VMEM((_TPW,), jnp.int32),
            pltpu.VMEM((_TPW,), jnp.int32),
            pltpu.VMEM((_TPW,), jnp.float32),
            pltpu.VMEM((_TPW,), jnp.float32),
            pltpu.VMEM((_TPW, _D), jnp.float32),
            pltpu.VMEM((_TPW // 2, _D), jnp.float32),
            pltpu.SemaphoreType.DMA,
        ],
    )
    return f(y, pos, probs_t)


def kernel(x, Wg, bg, W1, b1, W2, b2):
    eidx, probs = _gate(x, Wg, bg)
    ef = eidx.T.reshape(_A)
    a_sorted, pos, gsz = _route(x, ef)
    pmap = _tile_maps(gsz[:_E])
    w1b = W1.astype(jnp.bfloat16)
    w2b = W2.astype(jnp.bfloat16)
    y = _ffn(a_sorted, pmap, w1b, b1, w2b, b2)
    out = _combine(y, pos, probs.T.reshape(_A))
    return out


# speedup vs baseline: 1.7564x; 1.1915x over previous
import functools

import jax
import jax.numpy as jnp
from jax import lax
from jax.experimental import pallas as pl
from jax.experimental.pallas import tpu as pltpu
from jax.experimental.pallas import tpu_sc as plsc

_N = 2048
_D = 1024
_DFF = 2048
_G0, _G1 = 2, 4
_E = _G0 * _G1
_K = 2
_A = _N * _K
_TM = 256
_NT_X = _N // _TM
_NT_A = _A // _TM
_STEPS = _NT_A + _E - 1
_NEG = -1e30

_NC, _NS, _L = 2, 16, 16
_NW = _NC * _NS
_TPW = _N // _NW
_APW = _TPW * _K
_CPW = _APW // _L


def _gate_body(x_ref, wg_ref, bg_ref, eidx_ref, probs_ref, h0_ref, h1_ref):
    scores = jnp.dot(x_ref[...], wg_ref[...], preferred_element_type=jnp.float32)
    scores = scores + bg_ref[...]
    s0 = scores[:, :_G0]
    s1 = scores[:, _G0:_G0 + _G1]
    logits = jnp.concatenate([s0[:, i:i + 1] + s1 for i in range(_G0)], axis=1)
    ei = lax.broadcasted_iota(jnp.int32, (_TM, _E), 1)
    m1 = jnp.max(logits, axis=1, keepdims=True)
    a1 = jnp.min(jnp.where(logits == m1, ei, _E), axis=1, keepdims=True)
    l2 = jnp.where(ei == a1, _NEG, logits)
    m2 = jnp.max(l2, axis=1, keepdims=True)
    a2 = jnp.min(jnp.where(l2 == m2, ei, _E), axis=1, keepdims=True)
    z = jnp.exp(m2 - m1)
    p1 = 1.0 / (1.0 + z)
    p2 = z / (1.0 + z)
    eidx_ref[...] = jnp.concatenate([a1, a2], axis=1)
    probs_ref[...] = jnp.concatenate([p1, p2], axis=1)
    wpt = _TM // _TPW
    pad = jnp.zeros((wpt, _L - _E), jnp.int32)
    oh1 = (ei == a1).astype(jnp.int32).reshape(wpt, _TPW, _E)
    oh2 = (ei == a2).astype(jnp.int32).reshape(wpt, _TPW, _E)
    h0_ref[0] = jnp.concatenate([jnp.sum(oh1, axis=1), pad], axis=1)
    h1_ref[0] = jnp.concatenate([jnp.sum(oh2, axis=1), pad], axis=1)


def _gate(x, Wg, bg):
    gs = _G0 + _G1
    wpt = _TM // _TPW
    return pl.pallas_call(
        _gate_body,
        grid=(_NT_X,),
        in_specs=[
            pl.BlockSpec((_TM, _D), lambda t: (t, 0)),
            pl.BlockSpec((_D, gs), lambda t: (0, 0)),
            pl.BlockSpec((1, gs), lambda t: (0, 0)),
        ],
        out_specs=[
            pl.BlockSpec((_TM, _K), lambda t: (t, 0)),
            pl.BlockSpec((_TM, _K), lambda t: (t, 0)),
            pl.BlockSpec((1, wpt, _L), lambda t: (t, 0, 0)),
            pl.BlockSpec((1, wpt, _L), lambda t: (t, 0, 0)),
        ],
        out_shape=[
            jax.ShapeDtypeStruct((_N, _K), jnp.int32),
            jax.ShapeDtypeStruct((_N, _K), jnp.float32),
            jax.ShapeDtypeStruct((_NT_X, wpt, _L), jnp.int32),
            jax.ShapeDtypeStruct((_NT_X, wpt, _L), jnp.int32),
        ],
    )(x, Wg, bg.reshape(1, gs))


def _route_body(x_hbm, ef_hbm, h0_hbm, h1_hbm, a_hbm, pos_hbm, gs_hbm,
                efb_v, h0_v, h1_v, xrows_v, pe_v, po_v, gs_v, sem):
    wid = lax.axis_index("s") * _NC + lax.axis_index("c")
    iot = lax.iota(jnp.int32, _L)
    onehot = [(iot == e).astype(jnp.int32) for e in range(_E)]
    cpw_k = _TPW // _L

    pltpu.sync_copy(h0_hbm, h0_v)
    pltpu.sync_copy(h1_hbm, h1_v)
    pltpu.sync_copy(ef_hbm.at[pl.ds(wid * _TPW, _TPW)],
                    efb_v.at[pl.ds(0, _TPW)])
    pltpu.sync_copy(ef_hbm.at[pl.ds(_N + wid * _TPW, _TPW)],
                    efb_v.at[pl.ds(_TPW, _TPW)])

    def hist_step(i, carry):
        tot, tot0, pre0, pre1 = carry
        r0 = h0_v[i]
        r1 = h1_v[i]
        b = jnp.full((_L,), (i < wid).astype(jnp.int32))
        return (tot + r0 + r1, tot0 + r0, pre0 + r0 * b, pre1 + r1 * b)

    zeros = jnp.zeros((_L,), jnp.int32)
    tot, tot0, pre0, pre1 = lax.fori_loop(0, _NW, hist_step,
                                          (zeros, zeros, zeros, zeros))
    pre1 = pre1 + tot0

    excl = jnp.cumsum(tot) - tot
    base0 = excl + pre0
    base1 = excl + pre1

    ch0 = [efb_v[pl.ds(c * _L, _L)] for c in range(cpw_k)]
    ch1 = [efb_v[pl.ds(_TPW + c * _L, _L)] for c in range(cpw_k)]
    pc0 = [jnp.zeros((_L,), jnp.int32) for _ in range(cpw_k)]
    pc1 = [jnp.zeros((_L,), jnp.int32) for _ in range(cpw_k)]
    for e in range(_E):
        off0 = jnp.sum(jnp.where(onehot[e] == 1, base0, 0))
        off1 = jnp.sum(jnp.where(onehot[e] == 1, base1, 0))
        c0 = jnp.zeros((), jnp.int32)
        c1 = jnp.zeros((), jnp.int32)
        for c in range(cpw_k):
            m = ch0[c] == e
            rank = jnp.cumsum(m.astype(jnp.int32)) - 1
            pc0[c] = jnp.where(m, off0 + c0 + rank, pc0[c])
            c0 = c0 + jnp.sum(m.astype(jnp.int32))
            m = ch1[c] == e
            rank = jnp.cumsum(m.astype(jnp.int32)) - 1
            pc1[c] = jnp.where(m, off1 + c1 + rank, pc1[c])
            c1 = c1 + jnp.sum(m.astype(jnp.int32))
    for c in range(cpw_k):
        pe_v[pl.ds(c * _L, _L)] = pc0[c]
        po_v[pl.ds(c * _L, _L)] = pc1[c]

    pltpu.sync_copy(x_hbm.at[pl.ds(wid * _TPW, _TPW)], xrows_v)
    pltpu.async_copy(xrows_v, a_hbm.at[pe_v], sem).wait()
    pltpu.async_copy(xrows_v, a_hbm.at[po_v], sem).wait()

    pltpu.sync_copy(pe_v, pos_hbm.at[pl.ds(wid * _TPW, _TPW)])
    pltpu.sync_copy(po_v, pos_hbm.at[pl.ds(_N + wid * _TPW, _TPW)])

    @pl.when(wid == 0)
    def _():
        gs_v[...] = tot
        pltpu.sync_copy(gs_v, gs_hbm)


def _route(x, ef, h0, h1):
    mesh = plsc.VectorSubcoreMesh(core_axis_name="c", subcore_axis_name="s",
                                  num_cores=_NC, num_subcores=_NS)
    f = pl.kernel(
        _route_body,
        out_type=[
            jax.ShapeDtypeStruct((_A, _D), jnp.float32),
            jax.ShapeDtypeStruct((_A,), jnp.int32),
            jax.ShapeDtypeStruct((_L,), jnp.int32),
        ],
        mesh=mesh,
        compiler_params=pltpu.CompilerParams(needs_layout_passes=False),
        scratch_types=[
            pltpu.VMEM((_APW,), jnp.int32),
            pltpu.VMEM((_NW, _L), jnp.int32),
            pltpu.VMEM((_NW, _L), jnp.int32),
            pltpu.VMEM((_TPW, _D), jnp.float32),
            pltpu.VMEM((_TPW,), jnp.int32),
            pltpu.VMEM((_TPW,), jnp.int32),
            pltpu.VMEM((_L,), jnp.int32),
            pltpu.SemaphoreType.DMA,
        ],
    )
    return f(x, ef, h0, h1)


def _ffn_body(p_ref, a_ref, w1_ref, b1_ref, w2_ref, b2_ref, out_ref,
              w1b_ref, w2b_ref):
    s = pl.program_id(0)
    lo = p_ref[2, s]
    hi = p_ref[3, s]
    first = p_ref[4, s]

    @pl.when(p_ref[5, s] == 1)
    def _cast():
        w1b_ref[...] = w1_ref[0].astype(jnp.bfloat16)
        w2b_ref[...] = w2_ref[0].astype(jnp.bfloat16)

    a = a_ref[...].astype(jnp.bfloat16)
    h = jnp.dot(a, w1b_ref[...], preferred_element_type=jnp.float32)
    h = jnp.maximum(h + b1_ref[0], 0.0)
    y = jnp.dot(h.astype(jnp.bfloat16), w2b_ref[...],
                preferred_element_type=jnp.float32)
    y = y + b2_ref[0]
    ri = lax.broadcasted_iota(jnp.int32, (_TM, _D), 0)
    contrib = jnp.where((ri >= lo) & (ri < hi), y, 0.0)

    @pl.when(first == 1)
    def _init():
        out_ref[...] = contrib

    @pl.when(first == 0)
    def _acc():
        out_ref[...] = out_ref[...] + contrib


def _tile_maps(gsz):
    offs = jnp.concatenate([jnp.zeros((1,), jnp.int32), jnp.cumsum(gsz)])
    t = jnp.arange(_NT_A, dtype=jnp.int32)[:, None]
    e = jnp.arange(_E, dtype=jnp.int32)[None, :]
    seg_lo = jnp.maximum(offs[:-1][None, :], t * _TM)
    seg_hi = jnp.minimum(offs[1:][None, :], (t + 1) * _TM)
    active = seg_hi > seg_lo
    key = jnp.where(active, t * _E + e, 32767).reshape(-1)
    order = jnp.argsort(key)[:_STEPS]
    act = key[order] < 32767
    t23 = jnp.where(act, order // _E, _NT_A - 1).astype(jnp.int32)
    e23 = jnp.where(act, order % _E, _E - 1).astype(jnp.int32)
    lo23 = jnp.where(act, seg_lo.reshape(-1)[order] - t23 * _TM, 0)
    hi23 = jnp.where(act, seg_hi.reshape(-1)[order] - t23 * _TM, 0)
    prev_t = jnp.concatenate([jnp.full((1,), -1, jnp.int32), t23[:-1]])
    first = (act & (t23 != prev_t)).astype(jnp.int32)
    prev_e = jnp.concatenate([jnp.full((1,), -1, jnp.int32), e23[:-1]])
    wchange = (e23 != prev_e).astype(jnp.int32)
    return jnp.stack([t23, e23, lo23.astype(jnp.int32),
                      hi23.astype(jnp.int32), first, wchange])


def _ffn(a_sorted, pmap, W1, b1, W2, b2):
    grid_spec = pltpu.PrefetchScalarGridSpec(
        num_scalar_prefetch=1,
        grid=(_STEPS,),
        in_specs=[
            pl.BlockSpec((_TM, _D), lambda s, p: (p[0, s], 0)),
            pl.BlockSpec((1, _D, _DFF), lambda s, p: (p[1, s], 0, 0)),
            pl.BlockSpec((1, 1, _DFF), lambda s, p: (p[1, s], 0, 0)),
            pl.BlockSpec((1, _DFF, _D), lambda s, p: (p[1, s], 0, 0)),
            pl.BlockSpec((1, 1, _D), lambda s, p: (p[1, s], 0, 0)),
        ],
        out_specs=pl.BlockSpec((_TM, _D), lambda s, p: (p[0, s], 0)),
        scratch_shapes=[
            pltpu.VMEM((_D, _DFF), jnp.bfloat16),
            pltpu.VMEM((_DFF, _D), jnp.bfloat16),
        ],
    )
    return pl.pallas_call(
        _ffn_body,
        grid_spec=grid_spec,
        out_shape=jax.ShapeDtypeStruct((_A, _D), jnp.float32),
        compiler_params=pltpu.CompilerParams(
            dimension_semantics=("arbitrary",),
        ),
    )(pmap, a_sorted, W1, b1.reshape(_E, 1, _DFF), W2,
      b2.reshape(_E, 1, _D))


def _combine_body(y_hbm, pos_hbm, probs_hbm, out_hbm,
                  peb_v, pob_v, pr0_v, pr1_v, rows_v, outs_v, sem):
    wid = lax.axis_index("s") * _NC + lax.axis_index("c")
    pltpu.sync_copy(pos_hbm.at[pl.ds(wid * _TPW, _TPW)], peb_v)
    pltpu.sync_copy(pos_hbm.at[pl.ds(_N + wid * _TPW, _TPW)], pob_v)
    pltpu.sync_copy(probs_hbm.at[pl.ds(wid * _TPW, _TPW)], pr0_v)
    pltpu.sync_copy(probs_hbm.at[pl.ds(_N + wid * _TPW, _TPW)], pr1_v)
    half = _TPW // 2
    for hh in range(2):
        pltpu.async_copy(y_hbm.at[peb_v.at[pl.ds(hh * half, half)]],
                         rows_v.at[pl.ds(0, half)], sem).wait()
        pltpu.async_copy(y_hbm.at[pob_v.at[pl.ds(hh * half, half)]],
                         rows_v.at[pl.ds(half, half)], sem).wait()
        p0s, p1s = [], []
        for g in range(half // _L):
            v0 = pr0_v[pl.ds(hh * half + g * _L, _L)]
            v1 = pr1_v[pl.ds(hh * half + g * _L, _L)]
            for j in range(_L):
                p0s.append(v0[j])
                p1s.append(v1[j])

        def chunk_step(cc, _):
            sl = pl.ds(cc * _L, _L)
            for i in range(half):
                outs_v[i, sl] = (p0s[i] * rows_v[i, sl]
                                 + p1s[i] * rows_v[half + i, sl])
            return 0

        lax.fori_loop(0, _D // _L, chunk_step, 0)
        pltpu.sync_copy(outs_v,
                        out_hbm.at[pl.ds(wid * _TPW + hh * half, half)])


def _combine(y, pos, probs_t):
    mesh = plsc.VectorSubcoreMesh(core_axis_name="c", subcore_axis_name="s",
                                  num_cores=_NC, num_subcores=_NS)
    f = pl.kernel(
        _combine_body,
        out_type=jax.ShapeDtypeStruct((_N, _D), jnp.float32),
        mesh=mesh,
        compiler_params=pltpu.CompilerParams(needs_layout_passes=False),
        scratch_types=[
            pltpu.VMEM((_TPW,), jnp.int32),
            pltpu.VMEM((_TPW,), jnp.int32),
            pltpu.VMEM((_TPW,), jnp.float32),
            pltpu.VMEM((_TPW,), jnp.float32),
            pltpu.VMEM((_TPW, _D), jnp.float32),
            pltpu.VMEM((_TPW // 2, _D), jnp.float32),
            pltpu.SemaphoreType.DMA,
        ],
    )
    return f(y, pos, probs_t)


def kernel(x, Wg, bg, W1, b1, W2, b2):
    eidx, probs, h0, h1 = _gate(x, Wg, bg)
    ef = eidx.T.reshape(_A)
    a_sorted, pos, gsz = _route(x, ef, h0.reshape(_NW, _L),
                                h1.reshape(_NW, _L))
    pmap = _tile_maps(gsz[:_E])
    y = _ffn(a_sorted, pmap, W1, b1, W2, b2)
    out = _combine(y, pos, probs.T.reshape(_A))
    return out


# speedup vs baseline: 1.8375x; 1.0462x over previous
import functools

import jax
import jax.numpy as jnp
from jax import lax
from jax.experimental import pallas as pl
from jax.experimental.pallas import tpu as pltpu
from jax.experimental.pallas import tpu_sc as plsc

_N = 2048
_D = 1024
_DFF = 2048
_G0, _G1 = 2, 4
_E = _G0 * _G1
_K = 2
_A = _N * _K
_TM = 256
_NT_X = _N // _TM
_NT_A = _A // _TM
_STEPS = _NT_A + _E - 1
_NEG = -1e30

_NC, _NS, _L = 2, 16, 16
_NW = _NC * _NS
_TPW = _N // _NW
_APW = _TPW * _K
_CPW = _APW // _L


def _gate_body(x_ref, wg_ref, bg_ref, eidx_ref, probs_ref, h0_ref, h1_ref):
    scores = jnp.dot(x_ref[...], wg_ref[...], preferred_element_type=jnp.float32)
    scores = scores + bg_ref[...]
    s0 = scores[:, :_G0]
    s1 = scores[:, _G0:_G0 + _G1]
    logits = jnp.concatenate([s0[:, i:i + 1] + s1 for i in range(_G0)], axis=1)
    ei = lax.broadcasted_iota(jnp.int32, (_TM, _E), 1)
    m1 = jnp.max(logits, axis=1, keepdims=True)
    a1 = jnp.min(jnp.where(logits == m1, ei, _E), axis=1, keepdims=True)
    l2 = jnp.where(ei == a1, _NEG, logits)
    m2 = jnp.max(l2, axis=1, keepdims=True)
    a2 = jnp.min(jnp.where(l2 == m2, ei, _E), axis=1, keepdims=True)
    z = jnp.exp(m2 - m1)
    p1 = 1.0 / (1.0 + z)
    p2 = z / (1.0 + z)
    eidx_ref[...] = jnp.concatenate([a1, a2], axis=1)
    probs_ref[...] = jnp.concatenate([p1, p2], axis=1)
    wpt = _TM // _TPW
    pad = jnp.zeros((wpt, _L - _E), jnp.int32)
    oh1 = (ei == a1).astype(jnp.int32).reshape(wpt, _TPW, _E)
    oh2 = (ei == a2).astype(jnp.int32).reshape(wpt, _TPW, _E)
    h0_ref[0] = jnp.concatenate([jnp.sum(oh1, axis=1), pad], axis=1)
    h1_ref[0] = jnp.concatenate([jnp.sum(oh2, axis=1), pad], axis=1)


def _gate(x, Wg, bg):
    gs = _G0 + _G1
    wpt = _TM // _TPW
    return pl.pallas_call(
        _gate_body,
        grid=(_NT_X,),
        in_specs=[
            pl.BlockSpec((_TM, _D), lambda t: (t, 0)),
            pl.BlockSpec((_D, gs), lambda t: (0, 0)),
            pl.BlockSpec((1, gs), lambda t: (0, 0)),
        ],
        out_specs=[
            pl.BlockSpec((_TM, _K), lambda t: (t, 0)),
            pl.BlockSpec((_TM, _K), lambda t: (t, 0)),
            pl.BlockSpec((1, wpt, _L), lambda t: (t, 0, 0)),
            pl.BlockSpec((1, wpt, _L), lambda t: (t, 0, 0)),
        ],
        out_shape=[
            jax.ShapeDtypeStruct((_N, _K), jnp.int32),
            jax.ShapeDtypeStruct((_N, _K), jnp.float32),
            jax.ShapeDtypeStruct((_NT_X, wpt, _L), jnp.int32),
            jax.ShapeDtypeStruct((_NT_X, wpt, _L), jnp.int32),
        ],
    )(x, Wg, bg.reshape(1, gs))


def _route_body(x_hbm, ef_hbm, h0_hbm, h1_hbm, a_hbm, pos_hbm, gs_hbm,
                efb_v, h0_v, h1_v, xrows_v, pe_v, po_v, gs_v, sem):
    wid = lax.axis_index("s") * _NC + lax.axis_index("c")
    iot = lax.iota(jnp.int32, _L)
    onehot = [(iot == e).astype(jnp.int32) for e in range(_E)]
    cpw_k = _TPW // _L

    pltpu.sync_copy(h0_hbm, h0_v)
    pltpu.sync_copy(h1_hbm, h1_v)
    pltpu.sync_copy(ef_hbm.at[pl.ds(wid * _TPW, _TPW)],
                    efb_v.at[pl.ds(0, _TPW)])
    pltpu.sync_copy(ef_hbm.at[pl.ds(_N + wid * _TPW, _TPW)],
                    efb_v.at[pl.ds(_TPW, _TPW)])

    def hist_step(i, carry):
        tot, tot0, pre0, pre1 = carry
        r0 = h0_v[i]
        r1 = h1_v[i]
        b = jnp.full((_L,), (i < wid).astype(jnp.int32))
        return (tot + r0 + r1, tot0 + r0, pre0 + r0 * b, pre1 + r1 * b)

    zeros = jnp.zeros((_L,), jnp.int32)
    tot, tot0, pre0, pre1 = lax.fori_loop(0, _NW, hist_step,
                                          (zeros, zeros, zeros, zeros))
    pre1 = pre1 + tot0

    excl = jnp.cumsum(tot) - tot
    base0 = excl + pre0
    base1 = excl + pre1

    ch0 = [efb_v[pl.ds(c * _L, _L)] for c in range(cpw_k)]
    ch1 = [efb_v[pl.ds(_TPW + c * _L, _L)] for c in range(cpw_k)]
    pc0 = [jnp.zeros((_L,), jnp.int32) for _ in range(cpw_k)]
    pc1 = [jnp.zeros((_L,), jnp.int32) for _ in range(cpw_k)]
    for e in range(_E):
        off0 = jnp.sum(jnp.where(onehot[e] == 1, base0, 0))
        off1 = jnp.sum(jnp.where(onehot[e] == 1, base1, 0))
        c0 = jnp.zeros((), jnp.int32)
        c1 = jnp.zeros((), jnp.int32)
        for c in range(cpw_k):
            m = ch0[c] == e
            rank = jnp.cumsum(m.astype(jnp.int32)) - 1
            pc0[c] = jnp.where(m, off0 + c0 + rank, pc0[c])
            c0 = c0 + jnp.sum(m.astype(jnp.int32))
            m = ch1[c] == e
            rank = jnp.cumsum(m.astype(jnp.int32)) - 1
            pc1[c] = jnp.where(m, off1 + c1 + rank, pc1[c])
            c1 = c1 + jnp.sum(m.astype(jnp.int32))
    for c in range(cpw_k):
        pe_v[pl.ds(c * _L, _L)] = pc0[c]
        po_v[pl.ds(c * _L, _L)] = pc1[c]

    pltpu.sync_copy(x_hbm.at[pl.ds(wid * _TPW, _TPW)], xrows_v)
    pltpu.async_copy(xrows_v, a_hbm.at[pe_v], sem).wait()
    pltpu.async_copy(xrows_v, a_hbm.at[po_v], sem).wait()

    pltpu.sync_copy(pe_v, pos_hbm.at[pl.ds(wid * _TPW, _TPW)])
    pltpu.sync_copy(po_v, pos_hbm.at[pl.ds(_N + wid * _TPW, _TPW)])

    @pl.when(wid == 0)
    def _():
        gs_v[...] = tot
        pltpu.sync_copy(gs_v, gs_hbm)


def _route(x, ef, h0, h1):
    mesh = plsc.VectorSubcoreMesh(core_axis_name="c", subcore_axis_name="s",
                                  num_cores=_NC, num_subcores=_NS)
    f = pl.kernel(
        _route_body,
        out_type=[
            jax.ShapeDtypeStruct((_A, _D), jnp.float32),
            jax.ShapeDtypeStruct((_A,), jnp.int32),
            jax.ShapeDtypeStruct((_L,), jnp.int32),
        ],
        mesh=mesh,
        compiler_params=pltpu.CompilerParams(needs_layout_passes=False),
        scratch_types=[
            pltpu.VMEM((_APW,), jnp.int32),
            pltpu.VMEM((_NW, _L), jnp.int32),
            pltpu.VMEM((_NW, _L), jnp.int32),
            pltpu.VMEM((_TPW, _D), jnp.float32),
            pltpu.VMEM((_TPW,), jnp.int32),
            pltpu.VMEM((_TPW,), jnp.int32),
            pltpu.VMEM((_L,), jnp.int32),
            pltpu.SemaphoreType.DMA,
        ],
    )
    return f(x, ef, h0, h1)


def _ffn_body(p_ref, a_ref, w1_ref, b1_ref, w2_ref, b2_ref, out_ref,
              w1b_ref, w2b_ref):
    s = pl.program_id(0)
    lo = p_ref[2, s]
    hi = p_ref[3, s]
    first = p_ref[4, s]

    @pl.when(p_ref[5, s] == 1)
    def _cast():
        w1b_ref[...] = w1_ref[0].astype(jnp.bfloat16)
        w2b_ref[...] = w2_ref[0].astype(jnp.bfloat16)

    a = a_ref[...].astype(jnp.bfloat16)
    h = jnp.dot(a, w1b_ref[...], preferred_element_type=jnp.float32)
    h = jnp.maximum(h + b1_ref[0], 0.0)
    y = jnp.dot(h.astype(jnp.bfloat16), w2b_ref[...],
                preferred_element_type=jnp.float32)
    y = y + b2_ref[0]
    ri = lax.broadcasted_iota(jnp.int32, (_TM, _D), 0)
    contrib = jnp.where((ri >= lo) & (ri < hi), y, 0.0)

    @pl.when(first == 1)
    def _init():
        out_ref[...] = contrib

    @pl.when(first == 0)
    def _acc():
        out_ref[...] = out_ref[...] + contrib


def _tile_maps(gsz):
    offs = jnp.concatenate([jnp.zeros((1,), jnp.int32), jnp.cumsum(gsz)])
    t = jnp.arange(_NT_A, dtype=jnp.int32)[:, None]
    e = jnp.arange(_E, dtype=jnp.int32)[None, :]
    seg_lo = jnp.maximum(offs[:-1][None, :], t * _TM)
    seg_hi = jnp.minimum(offs[1:][None, :], (t + 1) * _TM)
    active = (seg_hi > seg_lo).reshape(-1)
    cidx = jnp.cumsum(active.astype(jnp.int32)) - 1
    dst = jnp.where(active, cidx, _STEPS)
    t_f = jnp.broadcast_to(t, (_NT_A, _E)).reshape(-1)
    e_f = jnp.broadcast_to(e, (_NT_A, _E)).reshape(-1)
    t23 = jnp.full((_STEPS,), _NT_A - 1, jnp.int32).at[dst].set(
        t_f, mode="drop")
    e23 = jnp.full((_STEPS,), _E - 1, jnp.int32).at[dst].set(
        e_f, mode="drop")
    lo_f = (seg_lo.reshape(-1) - t_f * _TM).astype(jnp.int32)
    hi_f = (seg_hi.reshape(-1) - t_f * _TM).astype(jnp.int32)
    lo23 = jnp.zeros((_STEPS,), jnp.int32).at[dst].set(lo_f, mode="drop")
    hi23 = jnp.zeros((_STEPS,), jnp.int32).at[dst].set(hi_f, mode="drop")
    act = hi23 > lo23
    prev_t = jnp.concatenate([jnp.full((1,), -1, jnp.int32), t23[:-1]])
    first = (act & (t23 != prev_t)).astype(jnp.int32)
    prev_e = jnp.concatenate([jnp.full((1,), -1, jnp.int32), e23[:-1]])
    wchange = (e23 != prev_e).astype(jnp.int32)
    return jnp.stack([t23, e23, lo23.astype(jnp.int32),
                      hi23.astype(jnp.int32), first, wchange])


def _ffn(a_sorted, pmap, W1, b1, W2, b2):
    grid_spec = pltpu.PrefetchScalarGridSpec(
        num_scalar_prefetch=1,
        grid=(_STEPS,),
        in_specs=[
            pl.BlockSpec((_TM, _D), lambda s, p: (p[0, s], 0)),
            pl.BlockSpec((1, _D, _DFF), lambda s, p: (p[1, s], 0, 0)),
            pl.BlockSpec((1, 1, _DFF), lambda s, p: (p[1, s], 0, 0)),
            pl.BlockSpec((1, _DFF, _D), lambda s, p: (p[1, s], 0, 0)),
            pl.BlockSpec((1, 1, _D), lambda s, p: (p[1, s], 0, 0)),
        ],
        out_specs=pl.BlockSpec((_TM, _D), lambda s, p: (p[0, s], 0)),
        scratch_shapes=[
            pltpu.VMEM((_D, _DFF), jnp.bfloat16),
            pltpu.VMEM((_DFF, _D), jnp.bfloat16),
        ],
    )
    return pl.pallas_call(
        _ffn_body,
        grid_spec=grid_spec,
        out_shape=jax.ShapeDtypeStruct((_A, _D), jnp.float32),
        compiler_params=pltpu.CompilerParams(
            dimension_semantics=("arbitrary",),
        ),
    )(pmap, a_sorted, W1, b1.reshape(_E, 1, _DFF), W2,
      b2.reshape(_E, 1, _D))


def _combine_body(y_hbm, pos_hbm, probs_hbm, out_hbm,
                  peb_v, pob_v, pr0_v, pr1_v, rows_v, outs_v,
                  gsem0, gsem1, wsem0, wsem1):
    wid = lax.axis_index("s") * _NC + lax.axis_index("c")
    pltpu.sync_copy(pos_hbm.at[pl.ds(wid * _TPW, _TPW)], peb_v)
    pltpu.sync_copy(pos_hbm.at[pl.ds(_N + wid * _TPW, _TPW)], pob_v)
    pltpu.sync_copy(probs_hbm.at[pl.ds(wid * _TPW, _TPW)], pr0_v)
    pltpu.sync_copy(probs_hbm.at[pl.ds(_N + wid * _TPW, _TPW)], pr1_v)
    grp = _L
    ngrp = _TPW // grp
    gsems = [gsem0, gsem1]
    wsems = [wsem0, wsem1]

    def start_gather(g, b):
        pltpu.async_copy(y_hbm.at[peb_v.at[pl.ds(g * grp, grp)]],
                         rows_v.at[b, pl.ds(0, grp)], gsems[b])
        pltpu.async_copy(y_hbm.at[pob_v.at[pl.ds(g * grp, grp)]],
                         rows_v.at[b, pl.ds(grp, grp)], gsems[b])

    start_gather(0, 0)
    start_gather(1, 1)
    writes = []
    for g in range(ngrp):
        b = g % 2
        pltpu.make_async_copy(y_hbm.at[peb_v.at[pl.ds(0, grp)]],
                              rows_v.at[b, pl.ds(0, grp)], gsems[b]).wait()
        pltpu.make_async_copy(y_hbm.at[pob_v.at[pl.ds(0, grp)]],
                              rows_v.at[b, pl.ds(grp, grp)], gsems[b]).wait()
        if g >= 2:
            writes[g - 2].wait()
        v0 = pr0_v[pl.ds(g * grp, _L)]
        v1 = pr1_v[pl.ds(g * grp, _L)]
        p0s = [v0[j] for j in range(_L)]
        p1s = [v1[j] for j in range(_L)]

        def chunk_step(cc, _):
            sl = pl.ds(cc * _L, _L)
            for i in range(grp):
                outs_v[b, i, sl] = (p0s[i] * rows_v[b, i, sl]
                                    + p1s[i] * rows_v[b, grp + i, sl])
            return 0

        lax.fori_loop(0, _D // _L, chunk_step, 0)
        if g + 2 < ngrp:
            start_gather(g + 2, b)
        writes.append(
            pltpu.async_copy(outs_v.at[b],
                             out_hbm.at[pl.ds(wid * _TPW + g * grp, grp)],
                             wsems[b]))
    writes[-2].wait()
    writes[-1].wait()


def _combine(y, pos, probs_t):
    mesh = plsc.VectorSubcoreMesh(core_axis_name="c", subcore_axis_name="s",
                                  num_cores=_NC, num_subcores=_NS)
    f = pl.kernel(
        _combine_body,
        out_type=jax.ShapeDtypeStruct((_N, _D), jnp.float32),
        mesh=mesh,
        compiler_params=pltpu.CompilerParams(needs_layout_passes=False),
        scratch_types=[
            pltpu.VMEM((_TPW,), jnp.int32),
            pltpu.VMEM((_TPW,), jnp.int32),
            pltpu.VMEM((_TPW,), jnp.float32),
            pltpu.VMEM((_TPW,), jnp.float32),
            pltpu.VMEM((2, 2 * _L, _D), jnp.float32),
            pltpu.VMEM((2, _L, _D), jnp.float32),
            pltpu.SemaphoreType.DMA,
            pltpu.SemaphoreType.DMA,
            pltpu.SemaphoreType.DMA,
            pltpu.SemaphoreType.DMA,
        ],
    )
    return f(y, pos, probs_t)


def kernel(x, Wg, bg, W1, b1, W2, b2):
    eidx, probs, h0, h1 = _gate(x, Wg, bg)
    ef = eidx.T.reshape(_A)
    a_sorted, pos, gsz = _route(x, ef, h0.reshape(_NW, _L),
                                h1.reshape(_NW, _L))
    pmap = _tile_maps(gsz[:_E])
    y = _ffn(a_sorted, pmap, W1, b1, W2, b2)
    out = _combine(y, pos, probs.T.reshape(_A))
    return out


# speedup vs baseline: 1.8396x; 1.0012x over previous
import functools

import jax
import jax.numpy as jnp
from jax import lax
from jax.experimental import pallas as pl
from jax.experimental.pallas import tpu as pltpu
from jax.experimental.pallas import tpu_sc as plsc

_N = 2048
_D = 1024
_DFF = 2048
_G0, _G1 = 2, 4
_E = _G0 * _G1
_K = 2
_A = _N * _K
_TM = 256
_NT_X = _N // _TM
_NT_A = _A // _TM
_STEPS = _NT_A + _E - 1
_NEG = -1e30

_NC, _NS, _L = 2, 16, 16
_NW = _NC * _NS
_TPW = _N // _NW
_APW = _TPW * _K
_CPW = _APW // _L


def _gate_body(x_ref, wg_ref, bg_ref, eidx_ref, probs_ref, h0_ref, h1_ref):
    scores = jnp.dot(x_ref[...], wg_ref[...], preferred_element_type=jnp.float32)
    scores = scores + bg_ref[...]
    s0 = scores[:, :_G0]
    s1 = scores[:, _G0:_G0 + _G1]
    logits = jnp.concatenate([s0[:, i:i + 1] + s1 for i in range(_G0)], axis=1)
    ei = lax.broadcasted_iota(jnp.int32, (_TM, _E), 1)
    m1 = jnp.max(logits, axis=1, keepdims=True)
    a1 = jnp.min(jnp.where(logits == m1, ei, _E), axis=1, keepdims=True)
    l2 = jnp.where(ei == a1, _NEG, logits)
    m2 = jnp.max(l2, axis=1, keepdims=True)
    a2 = jnp.min(jnp.where(l2 == m2, ei, _E), axis=1, keepdims=True)
    z = jnp.exp(m2 - m1)
    p1 = 1.0 / (1.0 + z)
    p2 = z / (1.0 + z)
    eidx_ref[...] = jnp.concatenate([a1, a2], axis=1)
    probs_ref[...] = jnp.concatenate([p1, p2], axis=1)
    wpt = _TM // _TPW
    pad = jnp.zeros((wpt, _L - _E), jnp.int32)
    oh1 = (ei == a1).astype(jnp.int32).reshape(wpt, _TPW, _E)
    oh2 = (ei == a2).astype(jnp.int32).reshape(wpt, _TPW, _E)
    h0_ref[0] = jnp.concatenate([jnp.sum(oh1, axis=1), pad], axis=1)
    h1_ref[0] = jnp.concatenate([jnp.sum(oh2, axis=1), pad], axis=1)


def _gate(x, Wg, bg):
    gs = _G0 + _G1
    wpt = _TM // _TPW
    return pl.pallas_call(
        _gate_body,
        grid=(_NT_X,),
        in_specs=[
            pl.BlockSpec((_TM, _D), lambda t: (t, 0)),
            pl.BlockSpec((_D, gs), lambda t: (0, 0)),
            pl.BlockSpec((1, gs), lambda t: (0, 0)),
        ],
        out_specs=[
            pl.BlockSpec((_TM, _K), lambda t: (t, 0)),
            pl.BlockSpec((_TM, _K), lambda t: (t, 0)),
            pl.BlockSpec((1, wpt, _L), lambda t: (t, 0, 0)),
            pl.BlockSpec((1, wpt, _L), lambda t: (t, 0, 0)),
        ],
        out_shape=[
            jax.ShapeDtypeStruct((_N, _K), jnp.int32),
            jax.ShapeDtypeStruct((_N, _K), jnp.float32),
            jax.ShapeDtypeStruct((_NT_X, wpt, _L), jnp.int32),
            jax.ShapeDtypeStruct((_NT_X, wpt, _L), jnp.int32),
        ],
    )(x, Wg, bg.reshape(1, gs))


def _route_body(x_hbm, ef_hbm, h0_hbm, h1_hbm, a_hbm, pos_hbm, gs_hbm,
                efb_v, h0_v, h1_v, xrows_v, pe_v, po_v, gs_v, sem):
    wid = lax.axis_index("s") * _NC + lax.axis_index("c")
    iot = lax.iota(jnp.int32, _L)
    onehot = [(iot == e).astype(jnp.int32) for e in range(_E)]
    cpw_k = _TPW // _L

    pltpu.sync_copy(h0_hbm, h0_v)
    pltpu.sync_copy(h1_hbm, h1_v)
    pltpu.sync_copy(ef_hbm.at[pl.ds(wid * _TPW, _TPW)],
                    efb_v.at[pl.ds(0, _TPW)])
    pltpu.sync_copy(ef_hbm.at[pl.ds(_N + wid * _TPW, _TPW)],
                    efb_v.at[pl.ds(_TPW, _TPW)])

    def hist_step(i, carry):
        tot, tot0, pre0, pre1 = carry
        r0 = h0_v[i]
        r1 = h1_v[i]
        b = jnp.full((_L,), (i < wid).astype(jnp.int32))
        return (tot + r0 + r1, tot0 + r0, pre0 + r0 * b, pre1 + r1 * b)

    zeros = jnp.zeros((_L,), jnp.int32)
    tot, tot0, pre0, pre1 = lax.fori_loop(0, _NW, hist_step,
                                          (zeros, zeros, zeros, zeros))
    pre1 = pre1 + tot0

    excl = jnp.cumsum(tot) - tot
    base0 = excl + pre0
    base1 = excl + pre1

    ch0 = [efb_v[pl.ds(c * _L, _L)] for c in range(cpw_k)]
    ch1 = [efb_v[pl.ds(_TPW + c * _L, _L)] for c in range(cpw_k)]
    pc0 = [jnp.zeros((_L,), jnp.int32) for _ in range(cpw_k)]
    pc1 = [jnp.zeros((_L,), jnp.int32) for _ in range(cpw_k)]
    for e in range(_E):
        off0 = jnp.sum(jnp.where(onehot[e] == 1, base0, 0))
        off1 = jnp.sum(jnp.where(onehot[e] == 1, base1, 0))
        c0 = jnp.zeros((), jnp.int32)
        c1 = jnp.zeros((), jnp.int32)
        for c in range(cpw_k):
            m = ch0[c] == e
            rank = jnp.cumsum(m.astype(jnp.int32)) - 1
            pc0[c] = jnp.where(m, off0 + c0 + rank, pc0[c])
            c0 = c0 + jnp.sum(m.astype(jnp.int32))
            m = ch1[c] == e
            rank = jnp.cumsum(m.astype(jnp.int32)) - 1
            pc1[c] = jnp.where(m, off1 + c1 + rank, pc1[c])
            c1 = c1 + jnp.sum(m.astype(jnp.int32))
    for c in range(cpw_k):
        pe_v[pl.ds(c * _L, _L)] = pc0[c]
        po_v[pl.ds(c * _L, _L)] = pc1[c]

    pltpu.sync_copy(x_hbm.at[pl.ds(wid * _TPW, _TPW)], xrows_v)
    pltpu.async_copy(xrows_v, a_hbm.at[pe_v], sem).wait()
    pltpu.async_copy(xrows_v, a_hbm.at[po_v], sem).wait()

    pltpu.sync_copy(pe_v, pos_hbm.at[pl.ds(wid * _TPW, _TPW)])
    pltpu.sync_copy(po_v, pos_hbm.at[pl.ds(_N + wid * _TPW, _TPW)])

    @pl.when(wid == 0)
    def _():
        gs_v[...] = tot
        pltpu.sync_copy(gs_v, gs_hbm)


def _route(x, ef, h0, h1):
    mesh = plsc.VectorSubcoreMesh(core_axis_name="c", subcore_axis_name="s",
                                  num_cores=_NC, num_subcores=_NS)
    f = pl.kernel(
        _route_body,
        out_type=[
            jax.ShapeDtypeStruct((_A, _D), jnp.float32),
            jax.ShapeDtypeStruct((_A,), jnp.int32),
            jax.ShapeDtypeStruct((_L,), jnp.int32),
        ],
        mesh=mesh,
        compiler_params=pltpu.CompilerParams(needs_layout_passes=False),
        scratch_types=[
            pltpu.VMEM((_APW,), jnp.int32),
            pltpu.VMEM((_NW, _L), jnp.int32),
            pltpu.VMEM((_NW, _L), jnp.int32),
            pltpu.VMEM((_TPW, _D), jnp.float32),
            pltpu.VMEM((_TPW,), jnp.int32),
            pltpu.VMEM((_TPW,), jnp.int32),
            pltpu.VMEM((_L,), jnp.int32),
            pltpu.SemaphoreType.DMA,
        ],
    )
    return f(x, ef, h0, h1)


def _ffn_body(p_ref, a_ref, w1_ref, b1_ref, w2_ref, b2_ref, out_ref,
              w1b_ref, w2b_ref):
    s = pl.program_id(0)
    lo = p_ref[2, s]
    hi = p_ref[3, s]
    first = p_ref[4, s]

    @pl.when(p_ref[5, s] == 1)
    def _cast():
        w1b_ref[...] = w1_ref[0].astype(jnp.bfloat16)
        w2b_ref[...] = w2_ref[0].astype(jnp.bfloat16)

    @pl.when(hi > lo)
    def _compute():
        a = a_ref[...].astype(jnp.bfloat16)
        h = jnp.dot(a, w1b_ref[...], preferred_element_type=jnp.float32)
        h = jnp.maximum(h + b1_ref[0], 0.0).astype(jnp.bfloat16)
        y = jnp.dot(h, w2b_ref[...], preferred_element_type=jnp.float32)
        y = y + b2_ref[0]
        ri = lax.broadcasted_iota(jnp.int32, (_TM, _D), 0)
        contrib = jnp.where((ri >= lo) & (ri < hi), y, 0.0)

        @pl.when(first == 1)
        def _init():
            out_ref[...] = contrib

        @pl.when(first == 0)
        def _acc():
            out_ref[...] = out_ref[...] + contrib


def _tile_maps(gsz):
    offs = jnp.concatenate([jnp.zeros((1,), jnp.int32), jnp.cumsum(gsz)])
    t = jnp.arange(_NT_A, dtype=jnp.int32)[:, None]
    e = jnp.arange(_E, dtype=jnp.int32)[None, :]
    seg_lo = jnp.maximum(offs[:-1][None, :], t * _TM)
    seg_hi = jnp.minimum(offs[1:][None, :], (t + 1) * _TM)
    active = (seg_hi > seg_lo).reshape(-1)
    cidx = jnp.cumsum(active.astype(jnp.int32)) - 1
    dst = jnp.where(active, cidx, _STEPS)
    t_f = jnp.broadcast_to(t, (_NT_A, _E)).reshape(-1)
    e_f = jnp.broadcast_to(e, (_NT_A, _E)).reshape(-1)
    t23 = jnp.full((_STEPS,), _NT_A - 1, jnp.int32).at[dst].set(
        t_f, mode="drop")
    e23 = jnp.full((_STEPS,), _E - 1, jnp.int32).at[dst].set(
        e_f, mode="drop")
    lo_f = (seg_lo.reshape(-1) - t_f * _TM).astype(jnp.int32)
    hi_f = (seg_hi.reshape(-1) - t_f * _TM).astype(jnp.int32)
    lo23 = jnp.zeros((_STEPS,), jnp.int32).at[dst].set(lo_f, mode="drop")
    hi23 = jnp.zeros((_STEPS,), jnp.int32).at[dst].set(hi_f, mode="drop")
    act = hi23 > lo23
    prev_t = jnp.concatenate([jnp.full((1,), -1, jnp.int32), t23[:-1]])
    first = (act & (t23 != prev_t)).astype(jnp.int32)
    prev_e = jnp.concatenate([jnp.full((1,), -1, jnp.int32), e23[:-1]])
    wchange = (e23 != prev_e).astype(jnp.int32)
    return jnp.stack([t23, e23, lo23.astype(jnp.int32),
                      hi23.astype(jnp.int32), first, wchange])


def _ffn(a_sorted, pmap, W1, b1, W2, b2):
    grid_spec = pltpu.PrefetchScalarGridSpec(
        num_scalar_prefetch=1,
        grid=(_STEPS,),
        in_specs=[
            pl.BlockSpec((_TM, _D), lambda s, p: (p[0, s], 0)),
            pl.BlockSpec((1, _D, _DFF), lambda s, p: (p[1, s], 0, 0)),
            pl.BlockSpec((1, 1, _DFF), lambda s, p: (p[1, s], 0, 0)),
            pl.BlockSpec((1, _DFF, _D), lambda s, p: (p[1, s], 0, 0)),
            pl.BlockSpec((1, 1, _D), lambda s, p: (p[1, s], 0, 0)),
        ],
        out_specs=pl.BlockSpec((_TM, _D), lambda s, p: (p[0, s], 0)),
        scratch_shapes=[
            pltpu.VMEM((_D, _DFF), jnp.bfloat16),
            pltpu.VMEM((_DFF, _D), jnp.bfloat16),
        ],
    )
    return pl.pallas_call(
        _ffn_body,
        grid_spec=grid_spec,
        out_shape=jax.ShapeDtypeStruct((_A, _D), jnp.float32),
        compiler_params=pltpu.CompilerParams(
            dimension_semantics=("arbitrary",),
        ),
    )(pmap, a_sorted, W1, b1.reshape(_E, 1, _DFF), W2,
      b2.reshape(_E, 1, _D))


def _combine_body(y_hbm, pos_hbm, probs_hbm, out_hbm,
                  peb_v, pob_v, pr0_v, pr1_v, rows_v, outs_v,
                  gsem0, gsem1, wsem0, wsem1):
    wid = lax.axis_index("s") * _NC + lax.axis_index("c")
    pltpu.sync_copy(pos_hbm.at[pl.ds(wid * _TPW, _TPW)], peb_v)
    pltpu.sync_copy(pos_hbm.at[pl.ds(_N + wid * _TPW, _TPW)], pob_v)
    pltpu.sync_copy(probs_hbm.at[pl.ds(wid * _TPW, _TPW)], pr0_v)
    pltpu.sync_copy(probs_hbm.at[pl.ds(_N + wid * _TPW, _TPW)], pr1_v)
    grp = _L
    ngrp = _TPW // grp
    gsems = [gsem0, gsem1]
    wsems = [wsem0, wsem1]

    def start_gather(g, b):
        pltpu.async_copy(y_hbm.at[peb_v.at[pl.ds(g * grp, grp)]],
                         rows_v.at[b, pl.ds(0, grp)], gsems[b])
        pltpu.async_copy(y_hbm.at[pob_v.at[pl.ds(g * grp, grp)]],
                         rows_v.at[b, pl.ds(grp, grp)], gsems[b])

    start_gather(0, 0)
    start_gather(1, 1)
    writes = []
    for g in range(ngrp):
        b = g % 2
        pltpu.make_async_copy(y_hbm.at[peb_v.at[pl.ds(0, grp)]],
                              rows_v.at[b, pl.ds(0, grp)], gsems[b]).wait()
        pltpu.make_async_copy(y_hbm.at[pob_v.at[pl.ds(0, grp)]],
                              rows_v.at[b, pl.ds(grp, grp)], gsems[b]).wait()
        if g >= 2:
            writes[g - 2].wait()
        v0 = pr0_v[pl.ds(g * grp, _L)]
        v1 = pr1_v[pl.ds(g * grp, _L)]
        p0s = [v0[j] for j in range(_L)]
        p1s = [v1[j] for j in range(_L)]

        def chunk_step(cc, _):
            sl = pl.ds(cc * _L, _L)
            for i in range(grp):
                outs_v[b, i, sl] = (p0s[i] * rows_v[b, i, sl]
                                    + p1s[i] * rows_v[b, grp + i, sl])
            return 0

        lax.fori_loop(0, _D // _L, chunk_step, 0)
        if g + 2 < ngrp:
            start_gather(g + 2, b)
        writes.append(
            pltpu.async_copy(outs_v.at[b],
                             out_hbm.at[pl.ds(wid * _TPW + g * grp, grp)],
                             wsems[b]))
    writes[-2].wait()
    writes[-1].wait()


def _combine(y, pos, probs_t):
    mesh = plsc.VectorSubcoreMesh(core_axis_name="c", subcore_axis_name="s",
                                  num_cores=_NC, num_subcores=_NS)
    f = pl.kernel(
        _combine_body,
        out_type=jax.ShapeDtypeStruct((_N, _D), jnp.float32),
        mesh=mesh,
        compiler_params=pltpu.CompilerParams(needs_layout_passes=False),
        scratch_types=[
            pltpu.VMEM((_TPW,), jnp.int32),
            pltpu.VMEM((_TPW,), jnp.int32),
            pltpu.VMEM((_TPW,), jnp.float32),
            pltpu.VMEM((_TPW,), jnp.float32),
            pltpu.VMEM((2, 2 * _L, _D), jnp.float32),
            pltpu.VMEM((2, _L, _D), jnp.float32),
            pltpu.SemaphoreType.DMA,
            pltpu.SemaphoreType.DMA,
            pltpu.SemaphoreType.DMA,
            pltpu.SemaphoreType.DMA,
        ],
    )
    return f(y, pos, probs_t)


def kernel(x, Wg, bg, W1, b1, W2, b2):
    eidx, probs, h0, h1 = _gate(x, Wg, bg)
    ef = eidx.T.reshape(_A)
    a_sorted, pos, gsz = _route(x, ef, h0.reshape(_NW, _L),
                                h1.reshape(_NW, _L))
    pmap = _tile_maps(gsz[:_E])
    y = _ffn(a_sorted, pmap, W1, b1, W2, b2)
    out = _combine(y, pos, probs.T.reshape(_A))
    return out
